# Initial kernel scaffold; baseline (speedup 1.0000x reference)
import jax
import jax.numpy as jnp
from jax.experimental import pallas as pl


def kernel(x, W1, b1, W2, b2, gamma):
    raise NotImplementedError("write your pallas kernel here")


# speedup vs baseline: 4.1679x; 4.1679x over previous
import functools

import jax
import jax.numpy as jnp
import numpy as np
from jax.experimental import pallas as pl
from jax.experimental.pallas import tpu as pltpu

_B, _C, _N = 4, 64, 1024
_U = 10


def _block_kernel(x_ref, w1_ref, b1_ref, w2_ref, b2_ref, gamma_ref, e_ref,
                  out_ref):
    xb = x_ref[0]
    q = xb.T

    dots = jax.lax.dot_general(q, q, (((1,), (1,)), ((), ())),
                               preferred_element_type=jnp.float32)
    qn = jnp.sqrt(jnp.sum(q * q, axis=1, keepdims=True))
    denom = jnp.maximum(qn * qn.T, 1e-8)
    adj = jnp.logical_or(dots > 0.5 * denom,
                         dots.T > 0.5 * denom).astype(jnp.float32)
    iota_r = jax.lax.broadcasted_iota(jnp.int32, (_N, _N), 0)
    iota_c = jax.lax.broadcasted_iota(jnp.int32, (_N, _N), 1)
    adj_i = adj + (iota_r == iota_c).astype(jnp.float32)
    rs = jnp.sum(adj_i, axis=1, keepdims=True)
    a_n = adj_i / rs

    qrs = jnp.sum(q, axis=1, keepdims=True)
    qr_inv = 1.0 / qrs
    qr_inv = jnp.where(jnp.isinf(qr_inv), 0.0, qr_inv)
    qf = q * qr_inv

    h1 = jnp.dot(qf, w1_ref[...], preferred_element_type=jnp.float32)
    h = jax.nn.relu(jnp.dot(a_n, h1, preferred_element_type=jnp.float32)
                    + b1_ref[...])
    h2 = jnp.dot(h, w2_ref[...], preferred_element_type=jnp.float32)
    queries = (jnp.dot(a_n, h2, preferred_element_type=jnp.float32)
               + b2_ref[...])

    m_max = None
    m_sum = None
    for s in range(_U):
        gs = jnp.dot(queries, e_ref[s], preferred_element_type=jnp.float32)
        qk = jnp.sum(queries * gs, axis=1, keepdims=True)
        m_max = qk if m_max is None else jnp.maximum(m_max, qk)
        m_sum = qk if m_sum is None else m_sum + qk
    m = m_max - m_sum * (1.0 / 64.0)

    iota_n = jax.lax.broadcasted_iota(jnp.int32, (_N, 1), 0)
    onehot_rows = []
    m_work = m
    for _ in range(_U):
        mv = jnp.max(m_work)
        cand = jnp.where(m_work == mv, iota_n, _N)
        sel = jnp.min(cand)
        row = (iota_n == sel)
        onehot_rows.append(row.astype(jnp.float32).T)
        m_work = jnp.where(row, -jnp.inf, m_work)
    o = jnp.concatenate(onehot_rows, axis=0)

    q_red = jnp.dot(o, queries, preferred_element_type=jnp.float32)
    scores = jax.lax.dot_general(q_red, queries, (((1,), (1,)), ((), ())),
                                 preferred_element_type=jnp.float32)
    scores = scores * (1.0 / np.sqrt(float(_N)))
    smax = jnp.max(scores, axis=1, keepdims=True)
    ex = jnp.exp(scores - smax)
    attn = ex / jnp.sum(ex, axis=1, keepdims=True)
    upd = jnp.dot(attn, queries, preferred_element_type=jnp.float32)

    ctx = queries
    shift = 1
    while shift < _N:
        ctx = ctx + jnp.concatenate(
            [jnp.zeros((shift, _C), jnp.float32), ctx[:-shift]], axis=0)
        shift *= 2
    mask = jnp.sum(o, axis=0, keepdims=True).T
    scat = jax.lax.dot_general(o, upd, (((0,), (0,)), ((), ())),
                               preferred_element_type=jnp.float32)
    ctx = jnp.where(mask > 0.0, scat, ctx)

    out_ref[0] = gamma_ref[0, 0] * ctx.T + xb


@functools.partial(jax.jit, static_argnames=())
def kernel(x, W1, b1, W2, b2, gamma):
    skey = jax.random.key(42)
    index_sample = jax.random.randint(skey, (_C, _U), 0, _C)
    e = (index_sample.T[:, None, :] ==
         jnp.arange(_C, dtype=index_sample.dtype)[None, :, None])
    e = e.astype(jnp.float32)

    grid = (_B,)
    out = pl.pallas_call(
        _block_kernel,
        grid=grid,
        in_specs=[
            pl.BlockSpec((1, _C, _N), lambda i: (i, 0, 0)),
            pl.BlockSpec((_C, 8), lambda i: (0, 0)),
            pl.BlockSpec((1, 8), lambda i: (0, 0)),
            pl.BlockSpec((8, _C), lambda i: (0, 0)),
            pl.BlockSpec((1, _C), lambda i: (0, 0)),
            pl.BlockSpec((1, 1), lambda i: (0, 0)),
            pl.BlockSpec((_U, _C, _C), lambda i: (0, 0, 0)),
        ],
        out_specs=pl.BlockSpec((1, _C, _N), lambda i: (i, 0, 0)),
        out_shape=jax.ShapeDtypeStruct((_B, _C, _N), jnp.float32),
        compiler_params=pltpu.CompilerParams(
            dimension_semantics=("arbitrary",),
        ),
    )(x, W1, b1.reshape(1, 8), W2, b2.reshape(1, _C),
      gamma.reshape(1, 1), e)
    return out


# speedup vs baseline: 5.5268x; 1.3260x over previous
import jax
import jax.numpy as jnp
import numpy as np
from jax.experimental import pallas as pl
from jax.experimental.pallas import tpu as pltpu

_B, _C, _N = 4, 64, 1024
_U = 10


def _block_kernel(x_ref, w1t_ref, b1_ref, w2t_ref, b2_ref, gamma_ref, e_ref,
                  out_ref):
    xb = x_ref[0]

    qn = jnp.sqrt(jnp.sum(xb * xb, axis=0, keepdims=True))
    qinv = jnp.where(qn > 0.0, 1.0 / qn, 0.0)
    qhat = xb * qinv
    sim = jax.lax.dot_general(qhat, qhat, (((0,), (0,)), ((), ())),
                              preferred_element_type=jnp.float32)
    adj = (sim > 0.5).astype(jnp.float32)
    rinv = 1.0 / (jnp.sum(adj, axis=0, keepdims=True) + 1.0)

    qrs_inv = 1.0 / jnp.sum(xb, axis=0, keepdims=True)
    qrs_inv = jnp.where(jnp.isinf(qrs_inv), 0.0, qrs_inv)
    qf = xb * qrs_inv

    h1 = jnp.dot(w1t_ref[...], qf, preferred_element_type=jnp.float32)
    p1 = jnp.dot(h1, adj, preferred_element_type=jnp.float32) + h1
    h = jax.nn.relu(p1 * rinv + b1_ref[...])
    h2 = jnp.dot(w2t_ref[...], h, preferred_element_type=jnp.float32)
    p2 = jnp.dot(h2, adj, preferred_element_type=jnp.float32) + h2
    queries = p2 * rinv + b2_ref[...]

    g = jnp.dot(e_ref[...], queries, preferred_element_type=jnp.float32)
    gq = g.reshape(_U, _C, _N) * queries[None, :, :]
    qk = jnp.sum(gq, axis=1)
    m = (jnp.max(qk, axis=0, keepdims=True)
         - jnp.sum(qk, axis=0, keepdims=True) * (1.0 / 64.0))

    iota_n = jax.lax.broadcasted_iota(jnp.int32, (1, _N), 1)
    onehot_rows = []
    m_work = m
    for _ in range(_U):
        mv = jnp.max(m_work)
        cand = jnp.where(m_work == mv, iota_n, _N)
        sel = jnp.min(cand)
        row = (iota_n == sel)
        onehot_rows.append(row.astype(jnp.float32))
        m_work = jnp.where(row, -jnp.inf, m_work)
    o = jnp.concatenate(onehot_rows, axis=0)

    q_red = jax.lax.dot_general(queries, o, (((1,), (1,)), ((), ())),
                                preferred_element_type=jnp.float32)
    scores = jax.lax.dot_general(q_red, queries, (((0,), (0,)), ((), ())),
                                 preferred_element_type=jnp.float32)
    scores = scores * (1.0 / np.sqrt(float(_N)))
    smax = jnp.max(scores, axis=1, keepdims=True)
    ex = jnp.exp(scores - smax)
    attn = ex / jnp.sum(ex, axis=1, keepdims=True)
    upd = jax.lax.dot_general(queries, attn, (((1,), (1,)), ((), ())),
                              preferred_element_type=jnp.float32)

    ctx = queries
    shift = 1
    while shift < _N:
        ctx = ctx + jnp.concatenate(
            [jnp.zeros((_C, shift), jnp.float32), ctx[:, :-shift]], axis=1)
        shift *= 2
    mask = jnp.sum(o, axis=0, keepdims=True)
    scat = jnp.dot(upd, o, preferred_element_type=jnp.float32)
    ctx = jnp.where(mask > 0.0, scat, ctx)

    out_ref[0] = gamma_ref[0, 0] * ctx + xb


def kernel(x, W1, b1, W2, b2, gamma):
    skey = jax.random.key(42)
    index_sample = jax.random.randint(skey, (_C, _U), 0, _C)
    e = (index_sample.T.reshape(_U * _C, 1) ==
         jnp.arange(_C, dtype=index_sample.dtype)[None, :])
    e = e.astype(jnp.float32)

    grid = (_B,)
    out = pl.pallas_call(
        _block_kernel,
        grid=grid,
        in_specs=[
            pl.BlockSpec((1, _C, _N), lambda i: (i, 0, 0)),
            pl.BlockSpec((8, _C), lambda i: (0, 0)),
            pl.BlockSpec((8, 1), lambda i: (0, 0)),
            pl.BlockSpec((_C, 8), lambda i: (0, 0)),
            pl.BlockSpec((_C, 1), lambda i: (0, 0)),
            pl.BlockSpec((1, 1), lambda i: (0, 0)),
            pl.BlockSpec((_U * _C, _C), lambda i: (0, 0)),
        ],
        out_specs=pl.BlockSpec((1, _C, _N), lambda i: (i, 0, 0)),
        out_shape=jax.ShapeDtypeStruct((_B, _C, _N), jnp.float32),
        compiler_params=pltpu.CompilerParams(
            dimension_semantics=("arbitrary",),
        ),
    )(x, W1.T, b1.reshape(8, 1), W2.T, b2.reshape(_C, 1),
      gamma.reshape(1, 1), e)
    return out
